# baseline (speedup 1.0000x reference)
import functools

import jax
import jax.numpy as jnp
from jax import lax
from jax.experimental import pallas as pl
from jax.experimental.pallas import tpu as pltpu
from jax.experimental.pallas import tpu_sc as plsc

N_NODES = 50000
D_IN = 512
NUM_SEGS = 512
NC = 2
NS = 16
NW = NC * NS
PER_W = 1568
CHUNK = 112
NCHUNK = PER_W // CHUNK
NVEC = D_IN // 16
UNROLL = 4
DUMMY = 513
EDGE0 = 640
OUT_ROWS = EDGE0 + 2 * NW
N_PAD = NW * PER_W


def _sc_segment_sums(x, ids):
    mesh = plsc.VectorSubcoreMesh(core_axis_name="c", subcore_axis_name="s")

    @functools.partial(
        pl.kernel,
        mesh=mesh,
        out_type=jax.ShapeDtypeStruct((OUT_ROWS * D_IN,), jnp.float32),
        scratch_types=[
            pltpu.VMEM((CHUNK, D_IN), jnp.float32),
            pltpu.VMEM((CHUNK, D_IN), jnp.float32),
            pltpu.VMEM((D_IN,), jnp.float32),
            pltpu.VMEM((CHUNK + 16,), jnp.int32),
            pltpu.VMEM((CHUNK + 16,), jnp.int32),
            pltpu.SemaphoreType.DMA,
            pltpu.SemaphoreType.DMA,
            pltpu.SemaphoreType.DMA,
            pltpu.SemaphoreType.DMA,
        ],
    )
    def k(x_hbm, ids_hbm, out_hbm, buf0, buf1, stage, idb0, idb1,
          sx0, sx1, si0, si1):
        bufs = (buf0, buf1)
        idbs = (idb0, idb1)
        sems = ((sx0, si0), (sx1, si1))
        c = lax.axis_index("c")
        s = lax.axis_index("s")
        wid = c * NS + s
        base = wid * PER_W
        e0row = EDGE0 + 2 * wid
        e1row = e0row + 1

        zero = jnp.zeros((16,), jnp.float32)
        for i in range(NVEC):
            stage[pl.ds(16 * i, 16)] = zero
        pltpu.sync_copy(stage, out_hbm.at[pl.ds(e0row * D_IN, D_IN)])
        pltpu.sync_copy(stage, out_hbm.at[pl.ds(e1row * D_IN, D_IN)])

        def make_row_body(buf, ids_sm):
            def one_row(j, carry):
                accs = carry[:NVEC]
                cur_id = carry[NVEC]
                nflush = carry[NVEC + 1]
                idj = ids_sm[pl.ds(j, 16)][0]
                same = idj == cur_id
                xs = [buf[j, pl.ds(16 * i, 16)] for i in range(NVEC)]

                @pl.when(jnp.logical_not(same))
                def _():
                    for i in range(NVEC):
                        stage[pl.ds(16 * i, 16)] = accs[i]
                    trow = jnp.where(
                        nflush == 0, DUMMY,
                        jnp.where(nflush == 1, e0row, cur_id))
                    pltpu.sync_copy(stage, out_hbm.at[pl.ds(trow * D_IN, D_IN)])

                nflush2 = jnp.where(same, nflush, nflush + 1)
                new_accs = tuple(
                    jnp.where(same, a + xv, xv) for a, xv in zip(accs, xs))
                return (*new_accs, idj, nflush2)

            def row_body(jj, carry):
                j = jj * UNROLL
                for u in range(UNROLL):
                    carry = one_row(j + u, carry)
                return carry
            return row_body

        row_bodies = (make_row_body(buf0, idb0), make_row_body(buf1, idb1))

        def issue(kk):
            first = base + kk * CHUNK
            start = jnp.minimum(first, N_NODES - CHUNK)
            p = kk % 2
            hx = pltpu.async_copy(x_hbm.at[pl.ds(start, CHUNK)],
                                  bufs[p], sems[p][0])
            hi = pltpu.async_copy(ids_hbm.at[pl.ds(start, CHUNK)],
                                  idbs[p].at[pl.ds(0, CHUNK)], sems[p][1])
            return first, start, hx, hi

        carry = (*([jnp.zeros((16,), jnp.float32)] * NVEC),
                 jnp.int32(-1), jnp.int32(0))
        pending = issue(0)
        for kk in range(NCHUNK):
            nxt = issue(kk + 1) if kk + 1 < NCHUNK else None
            first, start, hx, hi = pending
            hx.wait()
            hi.wait()
            carry = lax.fori_loop((first - start) // UNROLL, CHUNK // UNROLL,
                                  row_bodies[kk % 2], carry)
            pending = nxt

        accs = carry[:NVEC]
        nflush = carry[NVEC + 1]
        for i in range(NVEC):
            stage[pl.ds(16 * i, 16)] = accs[i]
        trow = jnp.where(nflush <= 1, e0row, e1row)
        pltpu.sync_copy(stage, out_hbm.at[pl.ds(trow * D_IN, D_IN)])

    return k(x, ids)


def _tc_counts(ids_mat):
    n_rows = ids_mat.shape[0]

    def body(i_ref, o_ref):
        seg = lax.broadcasted_iota(jnp.int32, (NUM_SEGS, 128), 0)

        def step(r, acc_c):
            row = i_ref[pl.ds(r, 1), :]
            return acc_c + jnp.where(seg == row, 1.0, 0.0)

        part = lax.fori_loop(
            0, n_rows, step, jnp.zeros((NUM_SEGS, 128), jnp.float32))
        o_ref[...] = jnp.sum(part, axis=1, keepdims=True)

    return pl.pallas_call(
        body,
        out_shape=jax.ShapeDtypeStruct((NUM_SEGS, 1), jnp.float32),
    )(ids_mat)


def _tc_head(acc, cnt_in, edge_ids, w, b2):
    d_out = w.shape[1]

    def body(a_ref, c_ref, e_ref, w_ref, b_ref, o_ref):
        cnt = c_ref[...]
        segc = lax.broadcasted_iota(jnp.int32, (NUM_SEGS, 2 * NW), 0)
        onehot_e = jnp.where(segc == e_ref[...], 1.0, 0.0)
        edge_sums = jnp.dot(onehot_e, a_ref[EDGE0:OUT_ROWS, :],
                            preferred_element_type=jnp.float32)
        is_edge = jnp.sum(onehot_e, axis=1, keepdims=True) > 0.0

        interior = jnp.where(cnt > 0.0, a_ref[:NUM_SEGS, :], 0.0)
        sums = jnp.where(is_edge, edge_sums, interior)
        emb = sums / jnp.maximum(cnt, 1.0)
        o_ref[...] = (jnp.dot(emb, w_ref[...], preferred_element_type=jnp.float32)
                      + b_ref[...])

    return pl.pallas_call(
        body,
        out_shape=jax.ShapeDtypeStruct((NUM_SEGS, d_out), jnp.float32),
    )(acc, cnt_in, edge_ids, w, b2)


def kernel(x, batch_ids, y, W, b):
    ids = batch_ids.astype(jnp.int32)
    sums_flat = _sc_segment_sums(x, ids)
    acc = sums_flat.reshape(OUT_ROWS, D_IN)
    ids_mat = jnp.concatenate(
        [ids, jnp.full((N_PAD - N_NODES,), NUM_SEGS, jnp.int32)]
    ).reshape(-1, 128)
    cnt = _tc_counts(ids_mat)
    pos = []
    for wd in range(NW):
        pos.append(wd * PER_W)
        pos.append(min((wd + 1) * PER_W, N_NODES) - 1)
    edge_ids = ids[jnp.array(pos, jnp.int32)].reshape(1, 2 * NW)
    pred = _tc_head(acc, cnt, edge_ids, W, b.reshape(1, -1))
    return (pred, y)


# speedup vs baseline: 1.6976x; 1.6976x over previous
import functools

import jax
import jax.numpy as jnp
from jax import lax
from jax.experimental import pallas as pl
from jax.experimental.pallas import tpu as pltpu
from jax.experimental.pallas import tpu_sc as plsc

N_NODES = 50000
D_IN = 512
NUM_SEGS = 512
NC = 2
NS = 16
NW = NC * NS
PER_W = 1568
CHUNK = 112
NCHUNK = PER_W // CHUNK
NVEC = D_IN // 16
UNROLL = 1
DUMMY = 513
EDGE0 = 640
OUT_ROWS = EDGE0 + 2 * NW
N_PAD = NW * PER_W


def _sc_segment_sums(x, ids):
    mesh = plsc.VectorSubcoreMesh(core_axis_name="c", subcore_axis_name="s")

    @functools.partial(
        pl.kernel,
        mesh=mesh,
        out_type=jax.ShapeDtypeStruct((OUT_ROWS * D_IN,), jnp.float32),
        scratch_types=[
            pltpu.VMEM((CHUNK, D_IN), jnp.float32),
            pltpu.VMEM((CHUNK, D_IN), jnp.float32),
            pltpu.VMEM((D_IN,), jnp.float32),
            pltpu.VMEM((CHUNK + 16,), jnp.int32),
            pltpu.VMEM((CHUNK + 16,), jnp.int32),
            pltpu.SemaphoreType.DMA,
            pltpu.SemaphoreType.DMA,
            pltpu.SemaphoreType.DMA,
            pltpu.SemaphoreType.DMA,
        ],
    )
    def k(x_hbm, ids_hbm, out_hbm, buf0, buf1, stage, idb0, idb1,
          sx0, sx1, si0, si1):
        bufs = (buf0, buf1)
        idbs = (idb0, idb1)
        sems = ((sx0, si0), (sx1, si1))
        c = lax.axis_index("c")
        s = lax.axis_index("s")
        wid = c * NS + s
        base = wid * PER_W
        e0row = EDGE0 + 2 * wid
        e1row = e0row + 1

        zero = jnp.zeros((16,), jnp.float32)
        for i in range(NVEC):
            stage[pl.ds(16 * i, 16)] = zero
        pltpu.sync_copy(stage, out_hbm.at[pl.ds(e0row * D_IN, D_IN)])
        pltpu.sync_copy(stage, out_hbm.at[pl.ds(e1row * D_IN, D_IN)])

        def make_row_body(buf, ids_sm):
            def one_row(j, carry):
                accs = carry[:NVEC]
                cur_id = carry[NVEC]
                nflush = carry[NVEC + 1]
                idj = ids_sm[pl.ds(j, 16)][0]
                same = idj == cur_id
                xs = [buf[j, pl.ds(16 * i, 16)] for i in range(NVEC)]

                @pl.when(jnp.logical_not(same))
                def _():
                    for i in range(NVEC):
                        stage[pl.ds(16 * i, 16)] = accs[i]
                    trow = jnp.where(
                        nflush == 0, DUMMY,
                        jnp.where(nflush == 1, e0row, cur_id))
                    pltpu.sync_copy(stage, out_hbm.at[pl.ds(trow * D_IN, D_IN)])

                nflush2 = jnp.where(same, nflush, nflush + 1)
                new_accs = tuple(
                    jnp.where(same, a + xv, xv) for a, xv in zip(accs, xs))
                return (*new_accs, idj, nflush2)

            def row_body(jj, carry):
                j = jj * UNROLL
                for u in range(UNROLL):
                    carry = one_row(j + u, carry)
                return carry
            return row_body

        row_bodies = (make_row_body(buf0, idb0), make_row_body(buf1, idb1))

        def issue(kk):
            first = base + kk * CHUNK
            start = jnp.minimum(first, N_NODES - CHUNK)
            p = kk % 2
            hx = pltpu.async_copy(x_hbm.at[pl.ds(start, CHUNK)],
                                  bufs[p], sems[p][0])
            hi = pltpu.async_copy(ids_hbm.at[pl.ds(start, CHUNK)],
                                  idbs[p].at[pl.ds(0, CHUNK)], sems[p][1])
            return first, start, hx, hi

        carry = (*([jnp.zeros((16,), jnp.float32)] * NVEC),
                 jnp.int32(-1), jnp.int32(0))
        pending = issue(0)
        for kk in range(NCHUNK):
            nxt = issue(kk + 1) if kk + 1 < NCHUNK else None
            first, start, hx, hi = pending
            hx.wait()
            hi.wait()
            carry = lax.fori_loop((first - start) // UNROLL, CHUNK // UNROLL,
                                  row_bodies[kk % 2], carry)
            pending = nxt

        accs = carry[:NVEC]
        nflush = carry[NVEC + 1]
        for i in range(NVEC):
            stage[pl.ds(16 * i, 16)] = accs[i]
        trow = jnp.where(nflush <= 1, e0row, e1row)
        pltpu.sync_copy(stage, out_hbm.at[pl.ds(trow * D_IN, D_IN)])

    return k(x, ids)


def _tc_counts(ids_mat):
    n_rows = ids_mat.shape[0]

    def body(i_ref, o_ref):
        seg = lax.broadcasted_iota(jnp.int32, (NUM_SEGS, 128), 0)

        def step(r, acc_c):
            row = i_ref[pl.ds(r, 1), :]
            return acc_c + jnp.where(seg == row, 1.0, 0.0)

        part = lax.fori_loop(
            0, n_rows, step, jnp.zeros((NUM_SEGS, 128), jnp.float32))
        o_ref[...] = jnp.sum(part, axis=1, keepdims=True)

    return pl.pallas_call(
        body,
        out_shape=jax.ShapeDtypeStruct((NUM_SEGS, 1), jnp.float32),
    )(ids_mat)


def _tc_head(acc, cnt_in, edge_ids, w, b2):
    d_out = w.shape[1]

    def body(a_ref, c_ref, e_ref, w_ref, b_ref, o_ref):
        cnt = c_ref[...]
        segc = lax.broadcasted_iota(jnp.int32, (NUM_SEGS, 2 * NW), 0)
        onehot_e = jnp.where(segc == e_ref[...], 1.0, 0.0)
        edge_sums = jnp.dot(onehot_e, a_ref[EDGE0:OUT_ROWS, :],
                            preferred_element_type=jnp.float32)
        is_edge = jnp.sum(onehot_e, axis=1, keepdims=True) > 0.0

        interior = jnp.where(cnt > 0.0, a_ref[:NUM_SEGS, :], 0.0)
        sums = jnp.where(is_edge, edge_sums, interior)
        emb = sums / jnp.maximum(cnt, 1.0)
        o_ref[...] = (jnp.dot(emb, w_ref[...], preferred_element_type=jnp.float32)
                      + b_ref[...])

    return pl.pallas_call(
        body,
        out_shape=jax.ShapeDtypeStruct((NUM_SEGS, d_out), jnp.float32),
    )(acc, cnt_in, edge_ids, w, b2)


def kernel(x, batch_ids, y, W, b):
    ids = batch_ids.astype(jnp.int32)
    sums_flat = _sc_segment_sums(x, ids)
    acc = sums_flat.reshape(OUT_ROWS, D_IN)
    ids_mat = jnp.concatenate(
        [ids, jnp.full((N_PAD - N_NODES,), NUM_SEGS, jnp.int32)]
    ).reshape(-1, 128)
    cnt = _tc_counts(ids_mat)
    pos = []
    for wd in range(NW):
        pos.append(wd * PER_W)
        pos.append(min((wd + 1) * PER_W, N_NODES) - 1)
    edge_ids = ids[jnp.array(pos, jnp.int32)].reshape(1, 2 * NW)
    pred = _tc_head(acc, cnt, edge_ids, W, b.reshape(1, -1))
    return (pred, y)
